# baseline (speedup 1.0000x reference)
import functools

import jax
import jax.numpy as jnp
from jax.experimental import pallas as pl
from jax.experimental.pallas import tpu as pltpu

_TOPK = 2
_ALPHA = 0.001
_W = 8
_TILE = 512


def _route_block(scores):
    eidx = jax.lax.broadcasted_iota(jnp.int32, scores.shape, 1)
    w1 = jnp.max(scores, axis=-1, keepdims=True)
    i1 = jnp.argmax(scores, axis=-1)[:, None]
    masked = jnp.where(eidx == i1, -1.0, scores)
    w2 = jnp.max(masked, axis=-1, keepdims=True)
    i2 = jnp.argmax(masked, axis=-1)[:, None]
    denom = w1 + w2 + 1e-20
    idx = jnp.concatenate([i1, i2], axis=1)
    wgt = jnp.concatenate([w1, w2], axis=1) / denom
    cnt = jnp.sum((eidx == i1).astype(jnp.float32)
                  + (eidx == i2).astype(jnp.float32), axis=0)
    ssum = jnp.sum(scores, axis=0)
    return idx, wgt, cnt, ssum


def _gate_kernel(*refs, g, bsz, seq_len, n_experts):
    hs_refs = refs[:_W]
    w_ref, idx_ref, wgt_ref, aux_ref, ce_acc, ss_acc = refs[_W:]
    i = pl.program_id(0)

    @pl.when(i == 0)
    def _init():
        ce_acc[...] = jnp.zeros_like(ce_acc)
        ss_acc[...] = jnp.zeros_like(ss_acc)

    w = w_ref[...]
    ce_new = ce_acc[...]
    ss_new = ss_acc[...]
    brange = jax.lax.broadcasted_iota(jnp.int32, (bsz, 1), 0)
    for k in range(_W):
        hs = hs_refs[k][...]
        logits = jax.lax.dot_general(hs, w, (((1,), (1,)), ((), ())),
                                     preferred_element_type=jnp.float32)
        m = jnp.max(logits, axis=-1, keepdims=True)
        unnorm = jnp.exp(logits - m)
        scores = unnorm / jnp.sum(unnorm, axis=-1, keepdims=True)
        idx, wgt, cnt, ssum = _route_block(scores)
        idx_ref[k] = idx
        wgt_ref[k] = wgt
        b = (i + k * g) * _TILE // seq_len
        bvec = (brange == b).astype(jnp.float32)
        ce_new = ce_new + bvec * cnt[None, :]
        ss_new = ss_new + bvec * ssum[None, :]
    ce_acc[...] = ce_new
    ss_acc[...] = ss_new

    @pl.when(i == g - 1)
    def _fin():
        ce = ce_acc[...] * (n_experts / (seq_len * _TOPK))
        mean_scores = ss_acc[...] / seq_len
        aux = jnp.sum(ce * mean_scores) / bsz * _ALPHA
        aux_ref[...] = jnp.broadcast_to(aux, (1, 1))


def kernel(hidden_states, weight):
    bsz, seq_len, dim = hidden_states.shape
    n_experts = weight.shape[0]
    n = bsz * seq_len
    hs = hidden_states.reshape(n, dim)
    g = n // _TILE // _W
    idx, wgt, aux = pl.pallas_call(
        functools.partial(_gate_kernel, g=g, bsz=bsz, seq_len=seq_len,
                          n_experts=n_experts),
        grid=(g,),
        in_specs=[
            pl.BlockSpec((_TILE, dim), lambda i, _k=k, _g=g: (i + _k * _g, 0))
            for k in range(_W)
        ] + [
            pl.BlockSpec((n_experts, dim), lambda i: (0, 0)),
        ],
        out_specs=(
            pl.BlockSpec((_W, _TILE, _TOPK), lambda i: (0, i, 0)),
            pl.BlockSpec((_W, _TILE, _TOPK), lambda i: (0, i, 0)),
            pl.BlockSpec((1, 1), lambda i: (0, 0)),
        ),
        out_shape=(
            jax.ShapeDtypeStruct((_W, n // _W, _TOPK), jnp.int32),
            jax.ShapeDtypeStruct((_W, n // _W, _TOPK), jnp.float32),
            jax.ShapeDtypeStruct((1, 1), jnp.float32),
        ),
        scratch_shapes=[
            pltpu.VMEM((bsz, n_experts), jnp.float32),
            pltpu.VMEM((bsz, n_experts), jnp.float32),
        ],
    )(*([hs] * _W), weight)
    return idx.reshape(n, _TOPK), wgt.reshape(n, _TOPK), aux[0, 0]


# speedup vs baseline: 2.0009x; 2.0009x over previous
import functools

import jax
import jax.numpy as jnp
from jax.experimental import pallas as pl
from jax.experimental.pallas import tpu as pltpu

_TOPK = 2
_ALPHA = 0.001
_W = 8
_TILE = 512


def _route_block(scores_t, n_experts):
    m1 = scores_t[0:1]
    i1 = jnp.zeros_like(m1, dtype=jnp.int32)
    for e in range(1, n_experts):
        s_e = scores_t[e:e + 1]
        take = s_e > m1
        m1 = jnp.where(take, s_e, m1)
        i1 = jnp.where(take, e, i1)
    m2 = jnp.full_like(m1, -1.0)
    i2 = jnp.zeros_like(m1, dtype=jnp.int32)
    for e in range(n_experts):
        s_e = scores_t[e:e + 1]
        take = (s_e > m2) & (i1 != e)
        m2 = jnp.where(take, s_e, m2)
        i2 = jnp.where(take, e, i2)
    den = m1 + m2 + 1e-20
    idx_t = jnp.concatenate([i1, i2], axis=0)
    wgt_t = jnp.concatenate([m1, m2], axis=0) / den
    eidx = jax.lax.broadcasted_iota(jnp.int32, scores_t.shape, 0)
    cnt = jnp.sum((eidx == i1).astype(jnp.float32)
                  + (eidx == i2).astype(jnp.float32),
                  axis=1, keepdims=True)
    ssum = jnp.sum(scores_t, axis=1, keepdims=True)
    return idx_t, wgt_t, cnt, ssum


def _gate_kernel(*refs, g, bsz, seq_len, n_experts):
    hs_refs = refs[:_W]
    w_ref, idx_ref, wgt_ref, aux_ref, ce_acc, ss_acc = refs[_W:]
    i = pl.program_id(0)

    @pl.when(i == 0)
    def _init():
        ce_acc[...] = jnp.zeros_like(ce_acc)
        ss_acc[...] = jnp.zeros_like(ss_acc)

    w = w_ref[...]
    ce_new = ce_acc[...]
    ss_new = ss_acc[...]
    brow = jax.lax.broadcasted_iota(jnp.int32, (1, bsz), 1)
    for k in range(_W):
        hs = hs_refs[k][...]
        logits_t = jax.lax.dot_general(w, hs, (((1,), (1,)), ((), ())),
                                       preferred_element_type=jnp.float32)
        m = jnp.max(logits_t, axis=0, keepdims=True)
        unnorm = jnp.exp(logits_t - m)
        scores_t = unnorm / jnp.sum(unnorm, axis=0, keepdims=True)
        idx_t, wgt_t, cnt, ssum = _route_block(scores_t, n_experts)
        idx_ref[k] = idx_t
        wgt_ref[k] = wgt_t
        b = (i + k * g) * _TILE // seq_len
        bvec = (brow == b).astype(jnp.float32)
        ce_new = ce_new + cnt * bvec
        ss_new = ss_new + ssum * bvec
    ce_acc[...] = ce_new
    ss_acc[...] = ss_new

    @pl.when(i == g - 1)
    def _fin():
        ce = ce_acc[...] * (n_experts / (seq_len * _TOPK))
        mean_scores = ss_acc[...] / seq_len
        aux = jnp.sum(ce * mean_scores) / bsz * _ALPHA
        aux_ref[...] = jnp.broadcast_to(aux, (1, 1))


def kernel(hidden_states, weight):
    bsz, seq_len, dim = hidden_states.shape
    n_experts = weight.shape[0]
    n = bsz * seq_len
    hs = hidden_states.reshape(n, dim)
    g = n // _TILE // _W
    idx_t, wgt_t, aux = pl.pallas_call(
        functools.partial(_gate_kernel, g=g, bsz=bsz, seq_len=seq_len,
                          n_experts=n_experts),
        grid=(g,),
        in_specs=[
            pl.BlockSpec((_TILE, dim), lambda i, _k=k, _g=g: (i + _k * _g, 0))
            for k in range(_W)
        ] + [
            pl.BlockSpec((n_experts, dim), lambda i: (0, 0)),
        ],
        out_specs=(
            pl.BlockSpec((_W, _TOPK, _TILE), lambda i: (0, 0, i)),
            pl.BlockSpec((_W, _TOPK, _TILE), lambda i: (0, 0, i)),
            pl.BlockSpec((1, 1), lambda i: (0, 0)),
        ),
        out_shape=(
            jax.ShapeDtypeStruct((_W, _TOPK, n // _W), jnp.int32),
            jax.ShapeDtypeStruct((_W, _TOPK, n // _W), jnp.float32),
            jax.ShapeDtypeStruct((1, 1), jnp.float32),
        ),
        scratch_shapes=[
            pltpu.VMEM((n_experts, bsz), jnp.float32),
            pltpu.VMEM((n_experts, bsz), jnp.float32),
        ],
    )(*([hs] * _W), weight)
    idx = idx_t.transpose(0, 2, 1).reshape(n, _TOPK)
    wgt = wgt_t.transpose(0, 2, 1).reshape(n, _TOPK)
    return idx, wgt, aux[0, 0]
